# baseline (speedup 1.0000x reference)
import jax
import jax.numpy as jnp
from jax import lax
from jax.experimental import pallas as pl
from jax.experimental.pallas import tpu as pltpu
from jax.experimental.pallas import tpu_sc as plsc

NUM_CLASSES = 21
V = 1024
N = 512
NC = 2
NS = 16
NW = NC * NS
ROWS_PW = N // NW
GROUPS = ROWS_PW * N // 16
VROWS_PS = 128
NBINS = 64


def _sc_body(outT_hbm, seg_hbm, tgt_hbm, hist_hbm,
             amx_v, lab64_v, lab_sh, lab_v, seg_v, tgt_v,
             hist_o, hist_t, hist_m, sem1, sem2):
    cid = lax.axis_index("c")
    sid = lax.axis_index("s")
    wid = sid * NC + cid
    row0 = wid * ROWS_PW
    cp1 = pltpu.async_copy(seg_hbm.at[pl.ds(row0, ROWS_PW), :], seg_v, sem1)
    cp2 = pltpu.async_copy(tgt_hbm.at[pl.ds(row0, ROWS_PW), :], tgt_v, sem2)

    @pl.when(sid < V // VROWS_PS)
    def _argmax():
        pltpu.sync_copy(outT_hbm.at[:, pl.ds(sid * VROWS_PS, VROWS_PS)],
                        amx_v)
        for g in range(VROWS_PS // 16):
            bv = amx_v[0, pl.ds(g * 16, 16)]
            bi = jnp.zeros((16,), jnp.int32)
            for c in range(1, NUM_CLASSES):
                v = amx_v[c, pl.ds(g * 16, 16)]
                upd = v > bv
                bv = jnp.where(upd, v, bv)
                bi = jnp.where(upd, jnp.full((16,), c, jnp.int32), bi)
            lab64_v[pl.ds(g * 16, 16)] = bi
        pltpu.sync_copy(lab64_v, lab_sh.at[pl.ds(sid * VROWS_PS, VROWS_PS)])

    plsc.subcore_barrier()
    pltpu.sync_copy(lab_sh, lab_v)

    zeros = jnp.zeros((16,), jnp.int32)
    for h in (hist_o, hist_t, hist_m):
        for b in range(NBINS):
            h[b, :] = zeros
    cp1.wait()
    cp2.wait()

    lanes = lax.iota(jnp.int32, 16)
    ones = jnp.ones((16,), jnp.int32)

    def body(i, carry):
        r = i >> 3
        c0 = (i & 7) << 6
        for u in range(4):
            col = c0 + u * 16
            seg = seg_v[r, pl.ds(col, 16)]
            tgt = tgt_v[r, pl.ds(col, 16)]
            lab = plsc.load_gather(lab_v, [seg])
            plsc.addupdate_scatter(hist_o, [lab, lanes], ones)
            plsc.addupdate_scatter(hist_t, [tgt, lanes], ones)
            plsc.addupdate_scatter(hist_m, [lab, lanes], ones,
                                   mask=lab == tgt)
        return carry

    lax.fori_loop(0, GROUPS // 4, body, 0)
    pltpu.sync_copy(hist_o, hist_hbm.at[wid, 0])
    pltpu.sync_copy(hist_t, hist_hbm.at[wid, 1])
    pltpu.sync_copy(hist_m, hist_hbm.at[wid, 2])


def _fin_body(hist_ref, out_ref):
    h = hist_ref[...].astype(jnp.float32)
    tot = jnp.sum(h, axis=(0, 3))
    o = tot[0:1, 0:NUM_CLASSES]
    t = tot[1:2, 0:NUM_CLASSES]
    m = tot[2:3, 0:NUM_CLASSES]
    score = (2.0 * m) / (o + t + 1e-10)
    out_ref[0, 0] = 1.0 - jnp.sum(score) / NUM_CLASSES


_sc_call = pl.kernel(
    _sc_body,
    out_type=jax.ShapeDtypeStruct((NW, 3, NBINS, 16), jnp.int32),
    mesh=plsc.VectorSubcoreMesh(core_axis_name="c", subcore_axis_name="s"),
    compiler_params=pltpu.CompilerParams(needs_layout_passes=False),
    scratch_types=[
        pltpu.VMEM((NUM_CLASSES, VROWS_PS), jnp.float32),
        pltpu.VMEM((VROWS_PS,), jnp.int32),
        pltpu.VMEM_SHARED((V,), jnp.int32),
        pltpu.VMEM((V,), jnp.int32),
        pltpu.VMEM((ROWS_PW, N), jnp.int32),
        pltpu.VMEM((ROWS_PW, N), jnp.int32),
        pltpu.VMEM((NBINS, 16), jnp.int32),
        pltpu.VMEM((NBINS, 16), jnp.int32),
        pltpu.VMEM((NBINS, 16), jnp.int32),
        pltpu.SemaphoreType.DMA,
        pltpu.SemaphoreType.DMA,
    ],
)


def kernel(output, target, segments):
    hist = _sc_call(output.T, segments, target)
    loss = pl.pallas_call(
        _fin_body,
        out_shape=jax.ShapeDtypeStruct((1, 1), jnp.float32),
        out_specs=pl.BlockSpec(memory_space=pltpu.SMEM),
    )(hist)
    return loss[0, 0]


# speedup vs baseline: 1.0075x; 1.0075x over previous
import jax
import jax.numpy as jnp
from jax import lax
from jax.experimental import pallas as pl
from jax.experimental.pallas import tpu as pltpu
from jax.experimental.pallas import tpu_sc as plsc

NUM_CLASSES = 21
V = 1024
N = 512
NC = 2
NS = 16
NW = NC * NS
ROWS_PW = N // NW
GROUPS = ROWS_PW * N // 16
VROWS_PS = 128
NBINS = 64


def _sc_body(outT_hbm, seg_hbm, tgt_hbm, hist_hbm,
             amx_v, lab64_v, lab_sh, lab_v, seg_v, tgt_v,
             hist_o, hist_t, hist_m, sem1, sem2):
    cid = lax.axis_index("c")
    sid = lax.axis_index("s")
    wid = sid * NC + cid
    row0 = wid * ROWS_PW
    cp1 = pltpu.async_copy(seg_hbm.at[pl.ds(row0, ROWS_PW), :], seg_v, sem1)
    cp2 = pltpu.async_copy(tgt_hbm.at[pl.ds(row0, ROWS_PW), :], tgt_v, sem2)

    @pl.when(sid < V // VROWS_PS)
    def _argmax():
        pltpu.sync_copy(outT_hbm.at[:, pl.ds(sid * VROWS_PS, VROWS_PS)],
                        amx_v)
        for g in range(VROWS_PS // 16):
            bv = amx_v[0, pl.ds(g * 16, 16)]
            bi = jnp.zeros((16,), jnp.int32)
            for c in range(1, NUM_CLASSES):
                v = amx_v[c, pl.ds(g * 16, 16)]
                upd = v > bv
                bv = jnp.where(upd, v, bv)
                bi = jnp.where(upd, jnp.full((16,), c, jnp.int32), bi)
            lab64_v[pl.ds(g * 16, 16)] = bi
        pltpu.sync_copy(lab64_v, lab_sh.at[pl.ds(sid * VROWS_PS, VROWS_PS)])

    plsc.subcore_barrier()
    pltpu.sync_copy(lab_sh, lab_v)

    zeros = jnp.zeros((16,), jnp.int32)
    for h in (hist_o, hist_t, hist_m):
        for b in range(NBINS):
            h[b, :] = zeros
    cp1.wait()
    cp2.wait()

    lanes = lax.iota(jnp.int32, 16)
    ones = jnp.ones((16,), jnp.int32)

    def body(i, carry):
        r = i >> 5
        col = (i & 31) * 16
        seg = seg_v[r, pl.ds(col, 16)]
        tgt = tgt_v[r, pl.ds(col, 16)]
        lab = plsc.load_gather(lab_v, [seg])
        plsc.addupdate_scatter(hist_o, [lab, lanes], ones)
        plsc.addupdate_scatter(hist_t, [tgt, lanes], ones)
        plsc.addupdate_scatter(hist_m, [lab, lanes], ones,
                               mask=lab == tgt)
        return carry

    lax.fori_loop(0, GROUPS, body, 0)
    pltpu.sync_copy(hist_o, hist_hbm.at[wid, 0])
    pltpu.sync_copy(hist_t, hist_hbm.at[wid, 1])
    pltpu.sync_copy(hist_m, hist_hbm.at[wid, 2])


def _fin_body(hist_ref, out_ref):
    h = hist_ref[...].astype(jnp.float32)
    tot = jnp.sum(h, axis=(0, 3))
    o = tot[0:1, 0:NUM_CLASSES]
    t = tot[1:2, 0:NUM_CLASSES]
    m = tot[2:3, 0:NUM_CLASSES]
    score = (2.0 * m) / (o + t + 1e-10)
    out_ref[0, 0] = 1.0 - jnp.sum(score) / NUM_CLASSES


_sc_call = pl.kernel(
    _sc_body,
    out_type=jax.ShapeDtypeStruct((NW, 3, NBINS, 16), jnp.int32),
    mesh=plsc.VectorSubcoreMesh(core_axis_name="c", subcore_axis_name="s"),
    compiler_params=pltpu.CompilerParams(needs_layout_passes=False),
    scratch_types=[
        pltpu.VMEM((NUM_CLASSES, VROWS_PS), jnp.float32),
        pltpu.VMEM((VROWS_PS,), jnp.int32),
        pltpu.VMEM_SHARED((V,), jnp.int32),
        pltpu.VMEM((V,), jnp.int32),
        pltpu.VMEM((ROWS_PW, N), jnp.int32),
        pltpu.VMEM((ROWS_PW, N), jnp.int32),
        pltpu.VMEM((NBINS, 16), jnp.int32),
        pltpu.VMEM((NBINS, 16), jnp.int32),
        pltpu.VMEM((NBINS, 16), jnp.int32),
        pltpu.SemaphoreType.DMA,
        pltpu.SemaphoreType.DMA,
    ],
)


def kernel(output, target, segments):
    hist = _sc_call(output.T, segments, target)
    loss = pl.pallas_call(
        _fin_body,
        out_shape=jax.ShapeDtypeStruct((1, 1), jnp.float32),
        out_specs=pl.BlockSpec(memory_space=pltpu.SMEM),
    )(hist)
    return loss[0, 0]


# speedup vs baseline: 1.1920x; 1.1832x over previous
import jax
import jax.numpy as jnp
from jax import lax
from jax.experimental import pallas as pl
from jax.experimental.pallas import tpu as pltpu
from jax.experimental.pallas import tpu_sc as plsc

NUM_CLASSES = 21
V = 1024
N = 512
NC = 2
NS = 16
NW = NC * NS
ROWS_PW = N // NW
GROUPS = ROWS_PW * N // 16
VROWS_PS = 128
NBINS = 64


def _sc_body(outT_hbm, seg_hbm, tgt_hbm, hist_hbm,
             amx_v, lab64_v, lab_sh, lab_v, seg_v, tgt_v,
             hist_o, hist_t, hist_m, sem1, sem2):
    cid = lax.axis_index("c")
    sid = lax.axis_index("s")
    wid = sid * NC + cid
    row0 = wid * ROWS_PW
    cp1 = pltpu.async_copy(seg_hbm.at[pl.ds(row0, ROWS_PW), :], seg_v, sem1)
    cp2 = pltpu.async_copy(tgt_hbm.at[pl.ds(row0, ROWS_PW), :], tgt_v, sem2)

    @pl.when(sid < V // VROWS_PS)
    def _argmax():
        pltpu.sync_copy(outT_hbm.at[:, pl.ds(sid * VROWS_PS, VROWS_PS)],
                        amx_v)
        for g in range(VROWS_PS // 16):
            bv = amx_v[0, pl.ds(g * 16, 16)]
            bi = jnp.zeros((16,), jnp.int32)
            for c in range(1, NUM_CLASSES):
                v = amx_v[c, pl.ds(g * 16, 16)]
                upd = v > bv
                bv = jnp.where(upd, v, bv)
                bi = jnp.where(upd, jnp.full((16,), c, jnp.int32), bi)
            lab64_v[pl.ds(g * 16, 16)] = bi
        pltpu.sync_copy(lab64_v, lab_sh.at[pl.ds(sid * VROWS_PS, VROWS_PS)])

    plsc.subcore_barrier()
    pltpu.sync_copy(lab_sh, lab_v)

    zeros = jnp.zeros((16,), jnp.int32)
    for h in (hist_o, hist_t, hist_m):
        for b in range(NBINS):
            h[b, :] = zeros
    cp1.wait()
    cp2.wait()

    lanes = lax.iota(jnp.int32, 16)
    ones = jnp.ones((16,), jnp.int32)

    @plsc.parallel_loop(0, GROUPS, unroll=4)
    def _hist_loop(i):
        r = i >> 5
        col = (i & 31) * 16
        seg = seg_v[r, pl.ds(col, 16)]
        tgt = tgt_v[r, pl.ds(col, 16)]
        lab = plsc.load_gather(lab_v, [seg])
        plsc.addupdate_scatter(hist_o, [lab, lanes], ones)
        plsc.addupdate_scatter(hist_t, [tgt, lanes], ones)
        plsc.addupdate_scatter(hist_m, [lab, lanes], ones,
                               mask=lab == tgt)
    pltpu.sync_copy(hist_o, hist_hbm.at[wid, 0])
    pltpu.sync_copy(hist_t, hist_hbm.at[wid, 1])
    pltpu.sync_copy(hist_m, hist_hbm.at[wid, 2])


def _fin_body(hist_ref, out_ref):
    h = hist_ref[...].astype(jnp.float32)
    tot = jnp.sum(h, axis=(0, 3))
    o = tot[0:1, 0:NUM_CLASSES]
    t = tot[1:2, 0:NUM_CLASSES]
    m = tot[2:3, 0:NUM_CLASSES]
    score = (2.0 * m) / (o + t + 1e-10)
    out_ref[0, 0] = 1.0 - jnp.sum(score) / NUM_CLASSES


_sc_call = pl.kernel(
    _sc_body,
    out_type=jax.ShapeDtypeStruct((NW, 3, NBINS, 16), jnp.int32),
    mesh=plsc.VectorSubcoreMesh(core_axis_name="c", subcore_axis_name="s"),
    compiler_params=pltpu.CompilerParams(needs_layout_passes=False),
    scratch_types=[
        pltpu.VMEM((NUM_CLASSES, VROWS_PS), jnp.float32),
        pltpu.VMEM((VROWS_PS,), jnp.int32),
        pltpu.VMEM_SHARED((V,), jnp.int32),
        pltpu.VMEM((V,), jnp.int32),
        pltpu.VMEM((ROWS_PW, N), jnp.int32),
        pltpu.VMEM((ROWS_PW, N), jnp.int32),
        pltpu.VMEM((NBINS, 16), jnp.int32),
        pltpu.VMEM((NBINS, 16), jnp.int32),
        pltpu.VMEM((NBINS, 16), jnp.int32),
        pltpu.SemaphoreType.DMA,
        pltpu.SemaphoreType.DMA,
    ],
)


def kernel(output, target, segments):
    hist = _sc_call(output.T, segments, target)
    loss = pl.pallas_call(
        _fin_body,
        out_shape=jax.ShapeDtypeStruct((1, 1), jnp.float32),
        out_specs=pl.BlockSpec(memory_space=pltpu.SMEM),
    )(hist)
    return loss[0, 0]
